# baseline (speedup 1.0000x reference)
import functools

import jax
import jax.numpy as jnp
from jax import lax
from jax.experimental import pallas as pl
from jax.experimental.pallas import tpu as pltpu
from jax.experimental.pallas import tpu_sc as plsc

_B = 16384
_F = 26
_V = 100000
_E = 16
_NC = 2
_NS = 16
_NW = _NC * _NS
_RW = _B // _NW
_RC = 16
_NCHUNK = _RW // _RC
_G = 4
_GI = _RC * _F // _G
_PW = _RW * _F


def _fm_body(fm_hbm, lin_hbm, idxh_hbm, idxf_hbm, col_hbm, dense_hbm, par_hbm,
             fin_hbm, like_hbm,
             idxh_v, idxf_v, col_v, dense_v, par_v, fm_buf, lin_buf, col_st,
             fin_v, like_v, sem):
  wid = lax.axis_index("s") * _NC + lax.axis_index("c")
  base = wid * _RW

  pltpu.sync_copy(idxh_hbm.at[pl.ds(wid * _PW, _PW)], idxh_v)
  pltpu.sync_copy(idxf_hbm.at[pl.ds(wid * _PW, _PW)], idxf_v)
  pltpu.sync_copy(col_hbm.at[pl.ds(wid * _PW, _PW)], col_v)
  pltpu.sync_copy(dense_hbm.at[pl.ds(wid * (_RW * 16 // 128), _RW * 16 // 128), :],
                  dense_v)
  pltpu.sync_copy(par_hbm, par_v)

  lanes = lax.iota(jnp.int32, 16)
  row0 = lanes * _F
  lane_hi = lax.shift_right_logical(lanes, 3)
  lane_col = lax.shift_left(lanes & 7, 4)

  w_d = [par_v[pl.ds(d * 16, 16)] for d in range(13)]
  b_dense = par_v[pl.ds(13 * 16, 16)]
  f_w, f_b = par_v[pl.ds(14 * 16, 16)], par_v[pl.ds(15 * 16, 16)]
  l_w, l_b = par_v[pl.ds(16 * 16, 16)], par_v[pl.ds(17 * 16, 16)]
  half = jnp.full((16,), 0.5, jnp.float32)

  def chunk_body(c, carry):
    copies = []
    for q in range(_G):
      copies.append(pltpu.async_copy(
          fm_hbm.at[idxh_v.at[pl.ds(c * (_RC * _F) + q * _GI, _GI)]],
          fm_buf.at[pl.ds(q * _GI, _GI), :], sem))
      copies.append(pltpu.async_copy(
          lin_hbm.at[idxf_v.at[pl.ds(c * (_RC * _F) + q * _GI, _GI)]],
          lin_buf.at[pl.ds(q * _GI, _GI)], sem))
    coff = c * (_RC * _F)
    for f in range(_F):
      col_st[pl.ds(f * 16, 16)] = plsc.load_gather(col_v, [coff + row0 + f])
    for cp in copies:
      cp.wait()

    second = jnp.zeros((16,), jnp.float32)
    for e in range(_E):
      esplat = jnp.full((16,), e, jnp.int32)
      acc = jnp.zeros((16,), jnp.float32)
      acc2 = jnp.zeros((16,), jnp.float32)
      for f in range(_F):
        v = plsc.load_gather(
            fm_buf, [row0 + f, col_st[pl.ds(f * 16, 16)] + esplat])
        acc = acc + v
        acc2 = acc2 + v * v
      second = second + (acc * acc - acc2)

    lin = jnp.zeros((16,), jnp.float32)
    for f in range(_F):
      lin = lin + plsc.load_gather(lin_buf, [row0 + f])

    dacc = b_dense
    drow = 2 * c + lane_hi
    for d in range(13):
      dv = plsc.load_gather(dense_v, [drow, lane_col + d])
      dacc = dacc + dv * w_d[d]

    logits = dacc + lin + half * second
    fin = 1.0 / (1.0 + jnp.exp(-(logits * f_w + f_b)))
    lik = 1.0 / (1.0 + jnp.exp(-(logits * l_w + l_b)))
    fin_v[pl.ds(c * _RC, _RC)] = fin
    like_v[pl.ds(c * _RC, _RC)] = lik
    return carry

  lax.fori_loop(0, _NCHUNK, chunk_body, 0)

  pltpu.sync_copy(fin_v, fin_hbm.at[pl.ds(base, _RW)])
  pltpu.sync_copy(like_v, like_hbm.at[pl.ds(base, _RW)])


@jax.jit
def _fm_mtl_sc(fm128, lin1d, idxh1d, idxf1d, col1d, dense128, params):
  mesh = plsc.VectorSubcoreMesh(
      core_axis_name="c", subcore_axis_name="s",
      num_cores=_NC, num_subcores=_NS)
  out = pl.kernel(
      _fm_body,
      out_type=(jax.ShapeDtypeStruct((_B,), jnp.float32),
                jax.ShapeDtypeStruct((_B,), jnp.float32)),
      mesh=mesh,
      scratch_types=[
          pltpu.VMEM((_PW,), jnp.int32),
          pltpu.VMEM((_PW,), jnp.int32),
          pltpu.VMEM((_PW,), jnp.int32),
          pltpu.VMEM((_RW * 16 // 128, 128), jnp.float32),
          pltpu.VMEM((18 * 16,), jnp.float32),
          pltpu.VMEM((_RC * _F, 128), jnp.float32),
          pltpu.VMEM((_RC * _F,), jnp.float32),
          pltpu.VMEM((_F * 16,), jnp.int32),
          pltpu.VMEM((_RW,), jnp.float32),
          pltpu.VMEM((_RW,), jnp.float32),
          pltpu.SemaphoreType.DMA,
      ],
      compiler_params=pltpu.CompilerParams(
          needs_layout_passes=False, use_tc_tiling_on_sc=True),
  )(fm128, lin1d, idxh1d, idxf1d, col1d, dense128, params)
  return out


def kernel(sparse_inputs, dense_inputs, linear_dense_W, linear_dense_b,
           linear_tables, fm_tables, finish_W, finish_b, like_W, like_b):
  fm128 = fm_tables.reshape(_F * _V * _E // 128, 128)
  lin1d = linear_tables.reshape(_F * _V)
  flat_idx = (sparse_inputs.astype(jnp.int32)
              + (jnp.arange(_F, dtype=jnp.int32) * _V)[None, :])
  idxh1d = (flat_idx >> 3).reshape(_B * _F)
  idxf1d = flat_idx.reshape(_B * _F)
  col1d = ((flat_idx & 7) << 4).reshape(_B * _F)
  dense128 = jnp.pad(dense_inputs, ((0, 0), (0, 3))).reshape(_B * 16 // 128, 128)
  p18 = jnp.concatenate([
      linear_dense_W.reshape(13), linear_dense_b.reshape(1),
      finish_W.reshape(1), finish_b.reshape(1),
      like_W.reshape(1), like_b.reshape(1),
  ])
  params = jnp.repeat(p18, 16)
  fin, lik = _fm_mtl_sc(fm128, lin1d, idxh1d, idxf1d, col1d, dense128, params)
  return fin.reshape(_B, 1), lik.reshape(_B, 1)


# speedup vs baseline: 1.0254x; 1.0254x over previous
import functools

import jax
import jax.numpy as jnp
from jax import lax
from jax.experimental import pallas as pl
from jax.experimental.pallas import tpu as pltpu
from jax.experimental.pallas import tpu_sc as plsc

_B = 16384
_F = 26
_V = 100000
_E = 16
_NC = 2
_NS = 16
_NW = _NC * _NS
_RW = _B // _NW
_RC = 16
_NCHUNK = _RW // _RC
_G = 4
_GI = _RC * _F // _G


def _fm_body(fm_hbm, lin_hbm, idx_hbm, dense_hbm, par_hbm,
             fin_hbm, like_hbm,
             idx_v, dense_v, par_v, fm_buf, lin_buf, fin_v, like_v, sem):
  wid = lax.axis_index("s") * _NC + lax.axis_index("c")
  base = wid * _RW

  pltpu.sync_copy(idx_hbm.at[pl.ds(wid * (_NCHUNK * _G), _NCHUNK * _G)], idx_v)
  pltpu.sync_copy(dense_hbm.at[pl.ds(base, _RW), :], dense_v)
  pltpu.sync_copy(par_hbm, par_v)

  lanes = lax.iota(jnp.int32, 16)
  row0 = lanes * _F

  w_d = [par_v[d, :] for d in range(13)]
  b_dense = par_v[13, :]
  f_w, f_b = par_v[14, :], par_v[15, :]
  l_w, l_b = par_v[16, :], par_v[17, :]
  half = jnp.full((16,), 0.5, jnp.float32)

  def chunk_body(c, carry):
    copies = []
    for q in range(_G):
      idx_row = idx_v.at[c * _G + q]
      copies.append(pltpu.async_copy(
          fm_hbm.at[idx_row], fm_buf.at[pl.ds(q * _GI, _GI), :], sem))
      copies.append(pltpu.async_copy(
          lin_hbm.at[idx_row], lin_buf.at[pl.ds(q * _GI, _GI)], sem))
    for cp in copies:
      cp.wait()

    second = jnp.zeros((16,), jnp.float32)
    for e in range(_E):
      col = jnp.full((16,), e, jnp.int32)
      acc = jnp.zeros((16,), jnp.float32)
      acc2 = jnp.zeros((16,), jnp.float32)
      for f in range(_F):
        v = plsc.load_gather(fm_buf, [row0 + f, col])
        acc = acc + v
        acc2 = acc2 + v * v
      second = second + (acc * acc - acc2)

    lin = jnp.zeros((16,), jnp.float32)
    for f in range(_F):
      lin = lin + plsc.load_gather(lin_buf, [row0 + f])

    dacc = b_dense
    drow = c * _RC + lanes
    for d in range(13):
      dv = plsc.load_gather(dense_v, [drow, jnp.full((16,), d, jnp.int32)])
      dacc = dacc + dv * w_d[d]

    logits = dacc + lin + half * second
    fin = 1.0 / (1.0 + jnp.exp(-(logits * f_w + f_b)))
    lik = 1.0 / (1.0 + jnp.exp(-(logits * l_w + l_b)))
    fin_v[pl.ds(c * _RC, _RC)] = fin
    like_v[pl.ds(c * _RC, _RC)] = lik
    return carry

  lax.fori_loop(0, _NCHUNK, chunk_body, 0)

  pltpu.sync_copy(fin_v, fin_hbm.at[pl.ds(base, _RW)])
  pltpu.sync_copy(like_v, like_hbm.at[pl.ds(base, _RW)])


@jax.jit
def _fm_mtl_sc(fm2d, lin1d, idx2d, dense_t, params):
  mesh = plsc.VectorSubcoreMesh(
      core_axis_name="c", subcore_axis_name="s",
      num_cores=_NC, num_subcores=_NS)
  out = pl.kernel(
      _fm_body,
      out_type=(jax.ShapeDtypeStruct((_B,), jnp.float32),
                jax.ShapeDtypeStruct((_B,), jnp.float32)),
      mesh=mesh,
      scratch_types=[
          pltpu.VMEM((_NCHUNK * _G, _GI), jnp.int32),
          pltpu.VMEM((_RW, 16), jnp.float32),
          pltpu.VMEM((18, 16), jnp.float32),
          pltpu.VMEM((_RC * _F, _E), jnp.float32),
          pltpu.VMEM((_RC * _F,), jnp.float32),
          pltpu.VMEM((_RW,), jnp.float32),
          pltpu.VMEM((_RW,), jnp.float32),
          pltpu.SemaphoreType.DMA,
      ],
      compiler_params=pltpu.CompilerParams(
          needs_layout_passes=False, use_tc_tiling_on_sc=False),
  )(fm2d, lin1d, idx2d, dense_t, params)
  return out


def kernel(sparse_inputs, dense_inputs, linear_dense_W, linear_dense_b,
           linear_tables, fm_tables, finish_W, finish_b, like_W, like_b):
  fm2d = fm_tables.reshape(_F * _V, _E)
  lin1d = linear_tables.reshape(_F * _V)
  flat_idx = (sparse_inputs.astype(jnp.int32)
              + (jnp.arange(_F, dtype=jnp.int32) * _V)[None, :])
  idx2d = flat_idx.reshape(_B * _F // _GI, _GI)
  dense16 = jnp.pad(dense_inputs, ((0, 0), (0, 3)))
  p18 = jnp.concatenate([
      linear_dense_W.reshape(13), linear_dense_b.reshape(1),
      finish_W.reshape(1), finish_b.reshape(1),
      like_W.reshape(1), like_b.reshape(1),
  ])
  params = jnp.tile(p18[:, None], (1, 16))
  fin, lik = _fm_mtl_sc(fm2d, lin1d, idx2d, dense16, params)
  return fin.reshape(_B, 1), lik.reshape(_B, 1)


# speedup vs baseline: 1.0593x; 1.0330x over previous
import functools

import jax
import jax.numpy as jnp
from jax import lax
from jax.experimental import pallas as pl
from jax.experimental.pallas import tpu as pltpu
from jax.experimental.pallas import tpu_sc as plsc

_B = 16384
_F = 26
_V = 100000
_E = 16
_NC = 2
_NS = 16
_NW = _NC * _NS
_RW = _B // _NW
_RC = 16
_NCHUNK = _RW // _RC
_G = 4
_GI = _RC * _F // _G


def _fm_body(fm_hbm, lin_hbm, idx_hbm, dense_hbm, par_hbm,
             fin_hbm, like_hbm,
             idx_v, dense_v, par_v, fm_buf0, lin_buf0, fm_buf1, lin_buf1,
             fin_v, like_v, sem0, sem1):
  wid = lax.axis_index("s") * _NC + lax.axis_index("c")
  base = wid * _RW

  pltpu.sync_copy(idx_hbm.at[pl.ds(wid * (_NCHUNK * _G), _NCHUNK * _G)], idx_v)
  pltpu.sync_copy(dense_hbm.at[pl.ds(base, _RW), :], dense_v)
  pltpu.sync_copy(par_hbm, par_v)

  lanes = lax.iota(jnp.int32, 16)
  row0 = lanes * _F

  w_d = [par_v[d, :] for d in range(13)]
  b_dense = par_v[13, :]
  f_w, f_b = par_v[14, :], par_v[15, :]
  l_w, l_b = par_v[16, :], par_v[17, :]
  half = jnp.full((16,), 0.5, jnp.float32)

  banks = ((fm_buf0, lin_buf0, sem0), (fm_buf1, lin_buf1, sem1))

  def fire(c, fmb, linb, sem):
    for q in range(_G):
      idx_row = idx_v.at[c * _G + q]
      pltpu.async_copy(fm_hbm.at[idx_row], fmb.at[pl.ds(q * _GI, _GI), :], sem)
      pltpu.async_copy(lin_hbm.at[idx_row], linb.at[pl.ds(q * _GI, _GI)], sem)

  def drain(fmb, linb, sem):
    for q in range(_G):
      idx_row = idx_v.at[0]
      pltpu.make_async_copy(
          fm_hbm.at[idx_row], fmb.at[pl.ds(q * _GI, _GI), :], sem).wait()
      pltpu.make_async_copy(
          lin_hbm.at[idx_row], linb.at[pl.ds(q * _GI, _GI)], sem).wait()

  def compute(c, fm_buf, lin_buf):
    second = jnp.zeros((16,), jnp.float32)
    for e in range(_E):
      col = jnp.full((16,), e, jnp.int32)
      acc = jnp.zeros((16,), jnp.float32)
      acc2 = jnp.zeros((16,), jnp.float32)
      for f in range(_F):
        v = plsc.load_gather(fm_buf, [row0 + f, col])
        acc = acc + v
        acc2 = acc2 + v * v
      second = second + (acc * acc - acc2)

    lin = jnp.zeros((16,), jnp.float32)
    for f in range(_F):
      lin = lin + plsc.load_gather(lin_buf, [row0 + f])

    dacc = b_dense
    drow = c * _RC + lanes
    for d in range(13):
      dv = plsc.load_gather(dense_v, [drow, jnp.full((16,), d, jnp.int32)])
      dacc = dacc + dv * w_d[d]

    logits = dacc + lin + half * second
    fin = 1.0 / (1.0 + jnp.exp(-(logits * f_w + f_b)))
    lik = 1.0 / (1.0 + jnp.exp(-(logits * l_w + l_b)))
    fin_v[pl.ds(c * _RC, _RC)] = fin
    like_v[pl.ds(c * _RC, _RC)] = lik

  fire(0, *banks[0])

  def pair_body(cc, carry):
    for b in range(2):
      c = cc * 2 + b

      @pl.when(c + 1 < _NCHUNK)
      def _():
        fire(c + 1, *banks[1 - b])

      drain(*banks[b])
      compute(c, banks[b][0], banks[b][1])
    return carry

  lax.fori_loop(0, _NCHUNK // 2, pair_body, 0)

  pltpu.sync_copy(fin_v, fin_hbm.at[pl.ds(base, _RW)])
  pltpu.sync_copy(like_v, like_hbm.at[pl.ds(base, _RW)])


@jax.jit
def _fm_mtl_sc(fm2d, lin1d, idx2d, dense_t, params):
  mesh = plsc.VectorSubcoreMesh(
      core_axis_name="c", subcore_axis_name="s",
      num_cores=_NC, num_subcores=_NS)
  out = pl.kernel(
      _fm_body,
      out_type=(jax.ShapeDtypeStruct((_B,), jnp.float32),
                jax.ShapeDtypeStruct((_B,), jnp.float32)),
      mesh=mesh,
      scratch_types=[
          pltpu.VMEM((_NCHUNK * _G, _GI), jnp.int32),
          pltpu.VMEM((_RW, 16), jnp.float32),
          pltpu.VMEM((18, 16), jnp.float32),
          pltpu.VMEM((_RC * _F, _E), jnp.float32),
          pltpu.VMEM((_RC * _F,), jnp.float32),
          pltpu.VMEM((_RC * _F, _E), jnp.float32),
          pltpu.VMEM((_RC * _F,), jnp.float32),
          pltpu.VMEM((_RW,), jnp.float32),
          pltpu.VMEM((_RW,), jnp.float32),
          pltpu.SemaphoreType.DMA,
          pltpu.SemaphoreType.DMA,
      ],
      compiler_params=pltpu.CompilerParams(
          needs_layout_passes=False, use_tc_tiling_on_sc=False),
  )(fm2d, lin1d, idx2d, dense_t, params)
  return out


def kernel(sparse_inputs, dense_inputs, linear_dense_W, linear_dense_b,
           linear_tables, fm_tables, finish_W, finish_b, like_W, like_b):
  fm2d = fm_tables.reshape(_F * _V, _E)
  lin1d = linear_tables.reshape(_F * _V)
  flat_idx = (sparse_inputs.astype(jnp.int32)
              + (jnp.arange(_F, dtype=jnp.int32) * _V)[None, :])
  idx2d = flat_idx.reshape(_B * _F // _GI, _GI)
  dense16 = jnp.pad(dense_inputs, ((0, 0), (0, 3)))
  p18 = jnp.concatenate([
      linear_dense_W.reshape(13), linear_dense_b.reshape(1),
      finish_W.reshape(1), finish_b.reshape(1),
      like_W.reshape(1), like_b.reshape(1),
  ])
  params = jnp.tile(p18[:, None], (1, 16))
  fin, lik = _fm_mtl_sc(fm2d, lin1d, idx2d, dense16, params)
  return fin.reshape(_B, 1), lik.reshape(_B, 1)
